# Initial kernel scaffold; baseline (speedup 1.0000x reference)
import jax
import jax.numpy as jnp
from jax.experimental import pallas as pl


def kernel(x, edge_index, batch, enc_W, enc_b, eps, W1, b1, W2, b2, gamma, beta, lin_W, lin_b):
    raise NotImplementedError("write your pallas kernel here")


# speedup vs baseline: 1.0845x; 1.0845x over previous
import functools

import jax
import jax.numpy as jnp
from jax import lax
from jax.experimental import pallas as pl

_BN_EPS = 1e-5


def _encoder_body(x_ref, w_ref, b_ref, o_ref):
    o_ref[...] = jnp.dot(x_ref[...], w_ref[...],
                         preferred_element_type=jnp.float32) + b_ref[...]


def _encoder(x, enc_W, enc_b):
    n, _ = x.shape
    h = enc_W.shape[1]
    return pl.pallas_call(
        _encoder_body,
        out_shape=jax.ShapeDtypeStruct((n, h), jnp.float32),
    )(x, enc_W, enc_b.reshape(1, h))


def _layer_body(h_ref, agg_ref, eps_ref, w1_ref, b1_ref, w2_ref, b2_ref,
                g_ref, be_ref, o_ref):
    h = h_ref[...]
    agg = agg_ref[0] + agg_ref[1]
    h2 = (1.0 + eps_ref[0, 0]) * h + agg
    t = jnp.maximum(jnp.dot(h2, w1_ref[...],
                            preferred_element_type=jnp.float32) + b1_ref[...],
                    0.0)
    h2 = jnp.dot(t, w2_ref[...], preferred_element_type=jnp.float32) + b2_ref[...]
    mean = jnp.mean(h2, axis=0, keepdims=True)
    c = h2 - mean
    var = jnp.mean(c * c, axis=0, keepdims=True)
    h2 = c * lax.rsqrt(var + _BN_EPS) * g_ref[...] + be_ref[...]
    o_ref[...] = jnp.maximum(h2, 0.0)


def _layer(h, agg, eps_i, W1_i, b1_i, W2_i, b2_i, gamma_i, beta_i):
    n, hd = h.shape
    return pl.pallas_call(
        _layer_body,
        out_shape=jax.ShapeDtypeStruct((n, hd), jnp.float32),
    )(h, agg, eps_i.reshape(1, 1), W1_i, b1_i.reshape(1, hd), W2_i,
      b2_i.reshape(1, hd), gamma_i.reshape(1, hd), beta_i.reshape(1, hd))


def _pool_body(h_ref, batch_ref, lw_ref, lb_ref, o_ref, *, g):
    n, hd = h_ref.shape
    b = batch_ref[...]
    gid = lax.broadcasted_iota(jnp.int32, (n, g), 1)
    onehot = jnp.where(b == gid, 1.0, 0.0)
    sums = lax.dot_general(onehot, h_ref[...], (((0,), (0,)), ((), ())),
                           preferred_element_type=jnp.float32)
    counts = jnp.sum(onehot, axis=0)[:, None]
    pooled = sums / jnp.maximum(counts, 1.0)
    o_ref[...] = jnp.dot(pooled, lw_ref[...],
                         preferred_element_type=jnp.float32) + lb_ref[...]


def _pool(h, batch, lin_W, lin_b, g):
    n, hd = h.shape
    c = lin_W.shape[1]
    return pl.pallas_call(
        functools.partial(_pool_body, g=g),
        out_shape=jax.ShapeDtypeStruct((g, c), jnp.float32),
    )(h, batch.reshape(n, 1), lin_W, lin_b.reshape(1, c))


def _segment_sum(h, src, dst, n):
    a = jax.ops.segment_sum(h[src], dst, num_segments=n)
    return jnp.stack([a, jnp.zeros_like(a)])


def kernel(x, edge_index, batch, enc_W, enc_b, eps, W1, b1, W2, b2, gamma,
           beta, lin_W, lin_b):
    n = x.shape[0]
    g = 64
    L = W1.shape[0]
    src = edge_index[0]
    dst = edge_index[1]
    h = _encoder(x, enc_W, enc_b)
    for i in range(L):
        agg = _segment_sum(h, src, dst, n)
        h = _layer(h, agg, eps[i], W1[i], b1[i], W2[i], b2[i], gamma[i],
                   beta[i])
    return _pool(h, batch, lin_W, lin_b, g)


# speedup vs baseline: 13.9797x; 12.8906x over previous
import functools

import jax
import jax.numpy as jnp
from jax import lax
from jax.experimental import pallas as pl
from jax.experimental.pallas import tpu as pltpu
from jax.experimental.pallas import tpu_sc as plsc

_BN_EPS = 1e-5

_NC = 2
_NS = 16


def _encoder_body(x_ref, w_ref, b_ref, o_ref):
    o_ref[...] = jnp.dot(x_ref[...], w_ref[...],
                         preferred_element_type=jnp.float32) + b_ref[...]


def _encoder(x, enc_W, enc_b):
    n, _ = x.shape
    h = enc_W.shape[1]
    return pl.pallas_call(
        _encoder_body,
        out_shape=jax.ShapeDtypeStruct((n, h), jnp.float32),
    )(x, enc_W, enc_b.reshape(1, h))


def _layer_body(h_ref, agg_ref, eps_ref, w1_ref, b1_ref, w2_ref, b2_ref,
                g_ref, be_ref, o_ref):
    h = h_ref[...]
    agg = agg_ref[0] + agg_ref[1]
    h2 = (1.0 + eps_ref[0, 0]) * h + agg
    t = jnp.maximum(jnp.dot(h2, w1_ref[...],
                            preferred_element_type=jnp.float32) + b1_ref[...],
                    0.0)
    h2 = jnp.dot(t, w2_ref[...], preferred_element_type=jnp.float32) + b2_ref[...]
    mean = jnp.mean(h2, axis=0, keepdims=True)
    c = h2 - mean
    var = jnp.mean(c * c, axis=0, keepdims=True)
    h2 = c * lax.rsqrt(var + _BN_EPS) * g_ref[...] + be_ref[...]
    o_ref[...] = jnp.maximum(h2, 0.0)


def _layer(h, agg, eps_i, W1_i, b1_i, W2_i, b2_i, gamma_i, beta_i):
    n, hd = h.shape
    return pl.pallas_call(
        _layer_body,
        out_shape=jax.ShapeDtypeStruct((n, hd), jnp.float32),
    )(h, agg, eps_i.reshape(1, 1), W1_i, b1_i.reshape(1, hd), W2_i,
      b2_i.reshape(1, hd), gamma_i.reshape(1, hd), beta_i.reshape(1, hd))


def _pool_body(h_ref, batch_ref, lw_ref, lb_ref, o_ref, *, g):
    n, hd = h_ref.shape
    b = batch_ref[...]
    gid = lax.broadcasted_iota(jnp.int32, (n, g), 1)
    onehot = jnp.where(b == gid, 1.0, 0.0)
    sums = lax.dot_general(onehot, h_ref[...], (((0,), (0,)), ((), ())),
                           preferred_element_type=jnp.float32)
    counts = jnp.sum(onehot, axis=0)[:, None]
    pooled = sums / jnp.maximum(counts, 1.0)
    o_ref[...] = jnp.dot(pooled, lw_ref[...],
                         preferred_element_type=jnp.float32) + lb_ref[...]


def _pool(h, batch, lin_W, lin_b, g):
    n, hd = h.shape
    c = lin_W.shape[1]
    return pl.pallas_call(
        functools.partial(_pool_body, g=g),
        out_shape=jax.ShapeDtypeStruct((g, c), jnp.float32),
    )(h, batch.reshape(n, 1), lin_W, lin_b.reshape(1, c))


_K = 125
_SLAB = 632


def _seg_body(h_hbm, src_hbm, dst_hbm, out_hbm,
              sidx, didx, rows, zbuf, accum, gsem, *, n, h, e):
    c = lax.axis_index("c")
    s = lax.axis_index("s")
    tile = c * _NS + s
    chunks = e // (_NC * _NS * _K)
    zrows = zbuf.shape[0]
    last = n - (_NS - 1) * _SLAB

    ebase = pl.multiple_of(tile * chunks, 8)
    pltpu.sync_copy(src_hbm.at[pl.ds(ebase, chunks)], sidx)
    pltpu.sync_copy(dst_hbm.at[pl.ds(ebase, chunks)], didx)

    z16 = jnp.zeros((16,), jnp.float32)
    for i in range(zrows):
        for j in range(h // 16):
            zbuf[i, pl.ds(j * 16, 16)] = z16
    row0 = pl.multiple_of(s * _SLAB, 8)
    myrows = jnp.where(s == _NS - 1, last, _SLAB)

    def zc(k, carry):
        pltpu.sync_copy(zbuf,
                        accum.at[pl.ds(pl.multiple_of(row0 + k * zrows, 8),
                                       zrows)])
        return carry
    lax.fori_loop(0, myrows // zrows, zc, 0)
    plsc.subcore_barrier()

    def start(g):
        slot = lax.rem(g, 2)
        pltpu.async_copy(h_hbm.at[sidx.at[g]], rows.at[slot], gsem.at[slot])

    start(0)
    start(1)

    def body(g, carry):
        slot = lax.rem(g, 2)
        pltpu.make_async_copy(h_hbm.at[sidx.at[g]], rows.at[slot],
                              gsem.at[slot]).wait()

        @pl.when(g + 2 < chunks)
        def _():
            start(g + 2)

        pltpu.sync_copy(rows.at[slot], accum.at[didx.at[g]], add=True)
        return carry
    lax.fori_loop(0, chunks, body, 0)
    plsc.subcore_barrier()

    @pl.when(s < _NS - 1)
    def _():
        pltpu.sync_copy(accum.at[pl.ds(row0, _SLAB)],
                        out_hbm.at[c, pl.ds(row0, _SLAB)])

    @pl.when(s == _NS - 1)
    def _():
        base = (_NS - 1) * _SLAB
        pltpu.sync_copy(accum.at[pl.ds(base, last)],
                        out_hbm.at[c, pl.ds(base, last)])


def _segment_sum(h, src2, dst2, n):
    e = src2.shape[0] * src2.shape[1]
    hd = h.shape[1]
    chunks = e // (_NC * _NS * _K)
    mesh = plsc.VectorSubcoreMesh(core_axis_name="c", subcore_axis_name="s")
    f = pl.kernel(
        functools.partial(_seg_body, n=n, h=hd, e=e),
        out_type=jax.ShapeDtypeStruct((_NC, n, hd), jnp.float32),
        mesh=mesh,
        scratch_types=[
            pltpu.VMEM((chunks, _K), jnp.int32),
            pltpu.VMEM((chunks, _K), jnp.int32),
            pltpu.VMEM((2, _K, hd), jnp.float32),
            pltpu.VMEM((8, hd), jnp.float32),
            pltpu.VMEM_SHARED((n, hd), jnp.float32),
            pltpu.SemaphoreType.DMA((2,)),
        ],
        compiler_params=pltpu.CompilerParams(use_tc_tiling_on_sc=False),
    )
    return f(h, src2, dst2)


def kernel(x, edge_index, batch, enc_W, enc_b, eps, W1, b1, W2, b2, gamma,
           beta, lin_W, lin_b):
    n = x.shape[0]
    g = 64
    L = W1.shape[0]
    e = edge_index.shape[1]
    src2 = edge_index[0].reshape(e // _K, _K)
    dst2 = edge_index[1].reshape(e // _K, _K)
    h = _encoder(x, enc_W, enc_b)
    for i in range(L):
        agg = _segment_sum(h, src2, dst2, n)
        h = _layer(h, agg, eps[i], W1[i], b1[i], W2[i], b2[i], gamma[i],
                   beta[i])
    return _pool(h, batch, lin_W, lin_b, g)


# speedup vs baseline: 14.3158x; 1.0240x over previous
import functools

import jax
import jax.numpy as jnp
from jax import lax
from jax.experimental import pallas as pl
from jax.experimental.pallas import tpu as pltpu
from jax.experimental.pallas import tpu_sc as plsc

_BN_EPS = 1e-5

_NC = 2
_NS = 16


def _encoder_body(x_ref, w_ref, b_ref, o_ref):
    o_ref[...] = jnp.dot(x_ref[...], w_ref[...],
                         preferred_element_type=jnp.float32) + b_ref[...]


def _encoder(x, enc_W, enc_b):
    n, _ = x.shape
    h = enc_W.shape[1]
    return pl.pallas_call(
        _encoder_body,
        out_shape=jax.ShapeDtypeStruct((n, h), jnp.float32),
    )(x, enc_W, enc_b.reshape(1, h))


def _layer_body(h_ref, agg_ref, eps_ref, w1_ref, b1_ref, w2_ref, b2_ref,
                g_ref, be_ref, o_ref):
    h = h_ref[...]
    agg = agg_ref[0] + agg_ref[1]
    h2 = (1.0 + eps_ref[0, 0]) * h + agg
    t = jnp.maximum(jnp.dot(h2, w1_ref[...],
                            preferred_element_type=jnp.float32) + b1_ref[...],
                    0.0)
    h2 = jnp.dot(t, w2_ref[...], preferred_element_type=jnp.float32) + b2_ref[...]
    mean = jnp.mean(h2, axis=0, keepdims=True)
    c = h2 - mean
    var = jnp.mean(c * c, axis=0, keepdims=True)
    h2 = c * lax.rsqrt(var + _BN_EPS) * g_ref[...] + be_ref[...]
    o_ref[...] = jnp.maximum(h2, 0.0)


def _layer(h, agg, eps_i, W1_i, b1_i, W2_i, b2_i, gamma_i, beta_i):
    n, hd = h.shape
    return pl.pallas_call(
        _layer_body,
        out_shape=jax.ShapeDtypeStruct((n, hd), jnp.float32),
    )(h, agg, eps_i.reshape(1, 1), W1_i, b1_i.reshape(1, hd), W2_i,
      b2_i.reshape(1, hd), gamma_i.reshape(1, hd), beta_i.reshape(1, hd))


def _layer_pool_body(h_ref, agg_ref, eps_ref, w1_ref, b1_ref, w2_ref,
                     b2_ref, g_ref, be_ref, batch_ref, lw_ref, lb_ref,
                     o_ref, *, g):
    n, hd = h_ref.shape
    h = h_ref[...]
    agg = agg_ref[0] + agg_ref[1]
    h2 = (1.0 + eps_ref[0, 0]) * h + agg
    t = jnp.maximum(jnp.dot(h2, w1_ref[...],
                            preferred_element_type=jnp.float32) + b1_ref[...],
                    0.0)
    h2 = jnp.dot(t, w2_ref[...], preferred_element_type=jnp.float32) + b2_ref[...]
    mean = jnp.mean(h2, axis=0, keepdims=True)
    c = h2 - mean
    var = jnp.mean(c * c, axis=0, keepdims=True)
    h2 = c * lax.rsqrt(var + _BN_EPS) * g_ref[...] + be_ref[...]
    hl = jnp.maximum(h2, 0.0)
    b = batch_ref[...]
    gid = lax.broadcasted_iota(jnp.int32, (n, g), 1)
    onehot = jnp.where(b == gid, 1.0, 0.0)
    sums = lax.dot_general(onehot, hl, (((0,), (0,)), ((), ())),
                           preferred_element_type=jnp.float32)
    counts = jnp.sum(onehot, axis=0)[:, None]
    pooled = sums / jnp.maximum(counts, 1.0)
    o_ref[...] = jnp.dot(pooled, lw_ref[...],
                         preferred_element_type=jnp.float32) + lb_ref[...]


def _layer_pool(h, agg, eps_i, W1_i, b1_i, W2_i, b2_i, gamma_i, beta_i,
                batch, lin_W, lin_b, g):
    n, hd = h.shape
    c = lin_W.shape[1]
    return pl.pallas_call(
        functools.partial(_layer_pool_body, g=g),
        out_shape=jax.ShapeDtypeStruct((g, c), jnp.float32),
    )(h, agg, eps_i.reshape(1, 1), W1_i, b1_i.reshape(1, hd), W2_i,
      b2_i.reshape(1, hd), gamma_i.reshape(1, hd), beta_i.reshape(1, hd),
      batch.reshape(n, 1), lin_W, lin_b.reshape(1, c))


_K = 125
_SLAB = 632


def _seg_body(h_hbm, src_hbm, dst_hbm, out_hbm,
              sidx, didx, rows, zbuf, accum, gsem, ssem, *, n, h, e):
    c = lax.axis_index("c")
    s = lax.axis_index("s")
    tile = c * _NS + s
    chunks = e // (_NC * _NS * _K)
    zrows = zbuf.shape[0]
    last = n - (_NS - 1) * _SLAB

    ebase = pl.multiple_of(tile * chunks, 8)
    pltpu.sync_copy(src_hbm.at[pl.ds(ebase, chunks)], sidx)
    pltpu.sync_copy(dst_hbm.at[pl.ds(ebase, chunks)], didx)

    z16 = jnp.zeros((16,), jnp.float32)
    for i in range(zrows):
        for j in range(h // 16):
            zbuf[i, pl.ds(j * 16, 16)] = z16
    row0 = pl.multiple_of(s * _SLAB, 8)
    myrows = jnp.where(s == _NS - 1, last, _SLAB)

    def zc(k, carry):
        pltpu.sync_copy(zbuf,
                        accum.at[pl.ds(pl.multiple_of(row0 + k * zrows, 8),
                                       zrows)])
        return carry
    lax.fori_loop(0, myrows // zrows, zc, 0)
    plsc.subcore_barrier()

    nbuf = rows.shape[0]

    def gstart(g):
        slot = lax.rem(g, nbuf)
        pltpu.async_copy(h_hbm.at[sidx.at[g]], rows.at[slot], gsem.at[slot])

    def swait(g):
        slot = lax.rem(g, nbuf)
        pltpu.make_async_copy(rows.at[slot], accum.at[didx.at[g]],
                              ssem.at[slot]).wait()

    gstart(0)
    gstart(1)
    gstart(2)

    def body(g, carry):
        slot = lax.rem(g, nbuf)
        pltpu.make_async_copy(h_hbm.at[sidx.at[g]], rows.at[slot],
                              gsem.at[slot]).wait()
        pltpu.async_copy(rows.at[slot], accum.at[didx.at[g]],
                         ssem.at[slot], add=True)

        @pl.when(g == 0)
        def _():
            gstart(3)

        @pl.when(jnp.logical_and(g >= 1, g + 3 < chunks))
        def _():
            swait(g - 1)
            gstart(g + 3)

        return carry
    lax.fori_loop(0, chunks, body, 0)
    swait(chunks - 4)
    swait(chunks - 3)
    swait(chunks - 2)
    swait(chunks - 1)
    plsc.subcore_barrier()

    @pl.when(s < _NS - 1)
    def _():
        pltpu.sync_copy(accum.at[pl.ds(row0, _SLAB)],
                        out_hbm.at[c, pl.ds(row0, _SLAB)])

    @pl.when(s == _NS - 1)
    def _():
        base = (_NS - 1) * _SLAB
        pltpu.sync_copy(accum.at[pl.ds(base, last)],
                        out_hbm.at[c, pl.ds(base, last)])


def _segment_sum(h, src2, dst2, n):
    e = src2.shape[0] * src2.shape[1]
    hd = h.shape[1]
    chunks = e // (_NC * _NS * _K)
    mesh = plsc.VectorSubcoreMesh(core_axis_name="c", subcore_axis_name="s")
    f = pl.kernel(
        functools.partial(_seg_body, n=n, h=hd, e=e),
        out_type=jax.ShapeDtypeStruct((_NC, n, hd), jnp.float32),
        mesh=mesh,
        scratch_types=[
            pltpu.VMEM((chunks, _K), jnp.int32),
            pltpu.VMEM((chunks, _K), jnp.int32),
            pltpu.VMEM((4, _K, hd), jnp.float32),
            pltpu.VMEM((8, hd), jnp.float32),
            pltpu.VMEM_SHARED((n, hd), jnp.float32),
            pltpu.SemaphoreType.DMA((4,)),
            pltpu.SemaphoreType.DMA((4,)),
        ],
        compiler_params=pltpu.CompilerParams(use_tc_tiling_on_sc=False),
    )
    return f(h, src2, dst2)


def kernel(x, edge_index, batch, enc_W, enc_b, eps, W1, b1, W2, b2, gamma,
           beta, lin_W, lin_b):
    n = x.shape[0]
    g = 64
    L = W1.shape[0]
    e = edge_index.shape[1]
    src2 = edge_index[0].reshape(e // _K, _K)
    dst2 = edge_index[1].reshape(e // _K, _K)
    h = _encoder(x, enc_W, enc_b)
    for i in range(L - 1):
        agg = _segment_sum(h, src2, dst2, n)
        h = _layer(h, agg, eps[i], W1[i], b1[i], W2[i], b2[i], gamma[i],
                   beta[i])
    agg = _segment_sum(h, src2, dst2, n)
    return _layer_pool(h, agg, eps[L - 1], W1[L - 1], b1[L - 1], W2[L - 1],
                       b2[L - 1], gamma[L - 1], beta[L - 1], batch, lin_W,
                       lin_b, g)


# speedup vs baseline: 15.8564x; 1.1076x over previous
import functools

import jax
import jax.numpy as jnp
from jax import lax
from jax.experimental import pallas as pl
from jax.experimental.pallas import tpu as pltpu
from jax.experimental.pallas import tpu_sc as plsc

_BN_EPS = 1e-5

_NC = 2
_NS = 16
_W = 128


def _encoder_body(x_ref, w_ref, b_ref, o_ref):
    o_ref[...] = jnp.dot(x_ref[...], w_ref[...],
                         preferred_element_type=jnp.float32) + b_ref[...]


def _encoder(x, enc_W, enc_b):
    n, _ = x.shape
    hd = enc_W.shape[1]
    return pl.pallas_call(
        _encoder_body,
        out_shape=jax.ShapeDtypeStruct((n, hd), jnp.float32),
    )(x, enc_W, enc_b.reshape(1, hd))


def _mlp_bn(h_ref, agg_ref, eps_ref, w1_ref, b1_ref, w2_ref, b2_ref,
            g_ref, be_ref, i, hd):
    h = h_ref[...]
    agg = agg_ref[0, :, :hd] + agg_ref[1, :, :hd]
    h2 = (1.0 + eps_ref[0, i]) * h + agg
    t = jnp.maximum(jnp.dot(h2, w1_ref[i],
                            preferred_element_type=jnp.float32)
                    + b1_ref[i][None, :], 0.0)
    h2 = (jnp.dot(t, w2_ref[i], preferred_element_type=jnp.float32)
          + b2_ref[i][None, :])
    mean = jnp.mean(h2, axis=0, keepdims=True)
    c = h2 - mean
    var = jnp.mean(c * c, axis=0, keepdims=True)
    h2 = c * lax.rsqrt(var + _BN_EPS) * g_ref[i][None, :] + be_ref[i][None, :]
    return jnp.maximum(h2, 0.0)


def _layer_body(h_ref, agg_ref, eps_ref, w1_ref, b1_ref, w2_ref, b2_ref,
                g_ref, be_ref, o_ref, *, i, hd):
    o_ref[...] = _mlp_bn(h_ref, agg_ref, eps_ref, w1_ref, b1_ref, w2_ref,
                         b2_ref, g_ref, be_ref, i, hd)


def _layer(h, agg, i, hd, eps2, W1, b1, W2, b2, gamma, beta):
    n = h.shape[0]
    return pl.pallas_call(
        functools.partial(_layer_body, i=i, hd=hd),
        out_shape=jax.ShapeDtypeStruct((n, hd), jnp.float32),
    )(h, agg, eps2, W1, b1, W2, b2, gamma, beta)


def _layer_pool_body(h_ref, agg_ref, eps_ref, w1_ref, b1_ref, w2_ref,
                     b2_ref, g_ref, be_ref, batch_ref, lw_ref, lb_ref,
                     o_ref, *, i, hd, g):
    n = h_ref.shape[0]
    hl = _mlp_bn(h_ref, agg_ref, eps_ref, w1_ref, b1_ref, w2_ref, b2_ref,
                 g_ref, be_ref, i, hd)
    b = batch_ref[...]
    gid = lax.broadcasted_iota(jnp.int32, (n, g), 1)
    onehot = jnp.where(b == gid, 1.0, 0.0)
    sums = lax.dot_general(onehot, hl, (((0,), (0,)), ((), ())),
                           preferred_element_type=jnp.float32)
    counts = jnp.sum(onehot, axis=0)[:, None]
    pooled = sums / jnp.maximum(counts, 1.0)
    o_ref[...] = jnp.dot(pooled, lw_ref[...],
                         preferred_element_type=jnp.float32) + lb_ref[...]


def _layer_pool(h, agg, i, hd, eps2, W1, b1, W2, b2, gamma, beta,
                batch, lin_W, lin_b, g):
    n = h.shape[0]
    c = lin_W.shape[1]
    return pl.pallas_call(
        functools.partial(_layer_pool_body, i=i, hd=hd, g=g),
        out_shape=jax.ShapeDtypeStruct((g, c), jnp.float32),
    )(h, agg, eps2, W1, b1, W2, b2, gamma, beta,
      batch.reshape(n, 1), lin_W, lin_b.reshape(1, c))


_K = 125
_SLAB = 632


def _seg_body(h_hbm, src_hbm, dst_hbm, out_hbm,
              sidx, didx, rows, zbuf, accum, gsem, ssem, *, n, h, e):
    c = lax.axis_index("c")
    s = lax.axis_index("s")
    tile = c * _NS + s
    chunks = e // (_NC * _NS * _K)
    zrows = zbuf.shape[0]
    last = n - (_NS - 1) * _SLAB

    ebase = pl.multiple_of(tile * chunks, 8)
    pltpu.sync_copy(src_hbm.at[pl.ds(ebase, chunks)], sidx)
    pltpu.sync_copy(dst_hbm.at[pl.ds(ebase, chunks)], didx)

    z16 = jnp.zeros((16,), jnp.float32)
    for i in range(zrows):
        for j in range(h // 16):
            zbuf[i, pl.ds(j * 16, 16)] = z16
    row0 = pl.multiple_of(s * _SLAB, 8)
    myrows = jnp.where(s == _NS - 1, last, _SLAB)

    def zc(k, carry):
        pltpu.sync_copy(zbuf,
                        accum.at[pl.ds(pl.multiple_of(row0 + k * zrows, 8),
                                       zrows)])
        return carry
    lax.fori_loop(0, myrows // zrows, zc, 0)
    plsc.subcore_barrier()

    nbuf = rows.shape[0]

    def gstart(g):
        slot = lax.rem(g, nbuf)
        pltpu.async_copy(h_hbm.at[sidx.at[g]], rows.at[slot], gsem.at[slot])

    def swait(g):
        slot = lax.rem(g, nbuf)
        pltpu.make_async_copy(rows.at[slot], accum.at[didx.at[g]],
                              ssem.at[slot]).wait()

    gstart(0)
    gstart(1)
    gstart(2)

    def body(g, carry):
        slot = lax.rem(g, nbuf)
        pltpu.make_async_copy(h_hbm.at[sidx.at[g]], rows.at[slot],
                              gsem.at[slot]).wait()
        pltpu.async_copy(rows.at[slot], accum.at[didx.at[g]],
                         ssem.at[slot], add=True)

        @pl.when(g == 0)
        def _():
            gstart(3)

        @pl.when(jnp.logical_and(g >= 1, g + 3 < chunks))
        def _():
            swait(g - 1)
            gstart(g + 3)

        return carry
    lax.fori_loop(0, chunks, body, 0)
    swait(chunks - 4)
    swait(chunks - 3)
    swait(chunks - 2)
    swait(chunks - 1)
    plsc.subcore_barrier()

    @pl.when(s < _NS - 1)
    def _():
        pltpu.sync_copy(accum.at[pl.ds(row0, _SLAB)],
                        out_hbm.at[c, pl.ds(row0, _SLAB), pl.ds(0, h)])

    @pl.when(s == _NS - 1)
    def _():
        base = (_NS - 1) * _SLAB
        pltpu.sync_copy(accum.at[pl.ds(base, last)],
                        out_hbm.at[c, pl.ds(base, last), pl.ds(0, h)])


def _segment_sum(h, src2, dst2, n, hd):
    e = src2.shape[0] * src2.shape[1]
    chunks = e // (_NC * _NS * _K)
    mesh = plsc.VectorSubcoreMesh(core_axis_name="c", subcore_axis_name="s")
    f = pl.kernel(
        functools.partial(_seg_body, n=n, h=hd, e=e),
        out_type=jax.ShapeDtypeStruct((_NC, n, _W), jnp.float32),
        mesh=mesh,
        scratch_types=[
            pltpu.VMEM((chunks, _K), jnp.int32),
            pltpu.VMEM((chunks, _K), jnp.int32),
            pltpu.VMEM((4, _K, hd), jnp.float32),
            pltpu.VMEM((8, hd), jnp.float32),
            pltpu.VMEM_SHARED((n, hd), jnp.float32),
            pltpu.SemaphoreType.DMA((4,)),
            pltpu.SemaphoreType.DMA((4,)),
        ],
        compiler_params=pltpu.CompilerParams(use_tc_tiling_on_sc=False),
    )
    return f(h, src2, dst2)


def kernel(x, edge_index, batch, enc_W, enc_b, eps, W1, b1, W2, b2, gamma,
           beta, lin_W, lin_b):
    n = x.shape[0]
    g = 64
    L = W1.shape[0]
    hd = enc_W.shape[1]
    e = edge_index.shape[1]
    src2 = edge_index[0].reshape(e // _K, _K)
    dst2 = edge_index[1].reshape(e // _K, _K)
    eps2 = eps.reshape(1, L)
    h = _encoder(x, enc_W, enc_b)
    for i in range(L - 1):
        agg = _segment_sum(h, src2, dst2, n, hd)
        h = _layer(h, agg, i, hd, eps2, W1, b1, W2, b2, gamma, beta)
    agg = _segment_sum(h, src2, dst2, n, hd)
    return _layer_pool(h, agg, L - 1, hd, eps2, W1, b1, W2, b2, gamma,
                       beta, batch, lin_W, lin_b, g)
